# baseline (speedup 1.0000x reference)
import jax
import jax.numpy as jnp
from jax import lax
from jax.experimental import pallas as pl
from jax.experimental.pallas import tpu as pltpu
from jax.experimental.pallas import tpu_sc as plsc

BN = 64
NB = 16
N_TILES = 4
N_TOKENS = 1025
D = 1280
ZERO_ROW = 16
TB = 513


def _sc_gather_body(hw_hbm, table_hbm, out_hbm, hw_v, idx_v, rows_v, sem):
    cid = lax.axis_index("c")
    sid = lax.axis_index("s")
    wid = sid * 2 + cid

    @pl.when(wid == 0)
    def _():
        pltpu.sync_copy(hw_hbm, hw_v)
        h = hw_v[pl.ds(0, NB)]
        w = hw_v[pl.ds(NB, NB)]
        n = jnp.where(w < 1, jnp.zeros_like(h), jnp.where(w == 1, h, h + h))
        for t in range(N_TILES):
            e1 = t * N_TILES
            e2 = (t // 2) * N_TILES + (t % 2)
            e = jnp.where(
                t < n,
                jnp.where(w >= 2, jnp.full((NB,), e2, jnp.int32),
                          jnp.full((NB,), e1, jnp.int32)),
                jnp.full((NB,), ZERO_ROW, jnp.int32),
            )
            idx_v[pl.ds(t * NB, NB)] = e
        pltpu.async_copy(table_hbm.at[idx_v], rows_v, sem).wait()
        pltpu.sync_copy(rows_v, out_hbm)


def _sc_gather(hw, table):
    mesh = plsc.VectorSubcoreMesh(core_axis_name="c", subcore_axis_name="s", num_cores=1)
    f = pl.kernel(
        _sc_gather_body,
        out_type=jax.ShapeDtypeStruct((BN, D), jnp.float32),
        mesh=mesh,
        scratch_types=[
            pltpu.VMEM((2 * NB,), jnp.int32),
            pltpu.VMEM((BN,), jnp.int32),
            pltpu.VMEM((BN, D), jnp.float32),
            pltpu.SemaphoreType.DMA,
        ],
    )
    return f(hw, table)


def _tc_add_body(gate_ref, x_ref, add_ref, o_ref):
    g = jnp.tanh(gate_ref[0])
    o_ref[...] = x_ref[...] + g * add_ref[:, 0, 0, :]


def _tc_add(gate, xt, addend):
    return pl.pallas_call(
        _tc_add_body,
        grid=(NB, (N_TOKENS + TB - 1) // TB),
        in_specs=[
            pl.BlockSpec(memory_space=pltpu.SMEM),
            pl.BlockSpec((1, TB, N_TILES, D), lambda b, c: (b, c, 0, 0)),
            pl.BlockSpec((N_TILES, 1, 1, D), lambda b, c: (0, b, 0, 0)),
        ],
        out_specs=pl.BlockSpec((1, TB, N_TILES, D), lambda b, c: (b, c, 0, 0)),
        out_shape=jax.ShapeDtypeStruct((NB, N_TOKENS, N_TILES, D), jnp.float32),
    )(gate, xt, addend)


def kernel(x, aspect_ratio, embedding, gate):
    bsz, n_tiles, n_tokens, d = x.shape
    ar = aspect_ratio.astype(jnp.int32)
    table = jnp.concatenate(
        [embedding.reshape(16, d), jnp.zeros((8, d), jnp.float32)], axis=0
    )
    hw = jnp.concatenate([ar[:, 0], ar[:, 1]])
    addend = _sc_gather(hw, table)
    xt = jnp.transpose(x, (0, 2, 1, 3))
    yt = _tc_add(gate, xt, addend.reshape(N_TILES, NB, 1, D))
    return jnp.transpose(yt, (0, 2, 1, 3))
